# Initial kernel scaffold; baseline (speedup 1.0000x reference)
import jax
import jax.numpy as jnp
from jax.experimental import pallas as pl


def kernel(x_transaction, x_client, edge_index_pays, edge_index_rev, Wl1_pt, bl1_pt, Wr1_pt, Wl1_tc, bl1_tc, Wr1_tc, Wl2_pt, bl2_pt, Wr2_pt, Wl2_tc, bl2_tc, Wr2_tc, Wd1, bd1, Wd2, bd2):
    raise NotImplementedError("write your pallas kernel here")


# speedup vs baseline: 8.0439x; 8.0439x over previous
import jax
import jax.numpy as jnp
from jax import lax
from jax.experimental import pallas as pl
from jax.experimental.pallas import tpu as pltpu
from jax.experimental.pallas import tpu_sc as plsc

N_NODES = 10000
N_PAD = 10240
N_EDGES = 320000
D = 128
NSUB = 16
CHUNK = 80
EDGES_PER_TILE = N_EDGES // NSUB
NCHUNK = EDGES_PER_TILE // CHUNK
ROWS_PER_TILE = N_PAD // NSUB

_mesh = plsc.VectorSubcoreMesh(core_axis_name="c", subcore_axis_name="s")


def _make_sc_aggregate(with_counts):
    n_out = 4 if with_counts else 2
    out_type = [jax.ShapeDtypeStruct((N_PAD, D), jnp.float32)] * n_out

    def body(xa, xb, sa, da, sb, db, zfeat, ones128, *rest):
        if with_counts:
            (agg_a, agg_b, cnt_a, cnt_b, acc_sh, ones_v,
             src_v0, src_v1, dst_v0, dst_v1, rows_v0, rows_v1,
             sem_i0, sem_i1, sem_g0, sem_g1) = rest
        else:
            (agg_a, agg_b, acc_sh, src_v0, src_v1, dst_v0, dst_v1,
             rows_v0, rows_v1, sem_i0, sem_i1, sem_g0, sem_g1) = rest
            cnt_a = cnt_b = None
        src_v = (src_v0, src_v1)
        dst_v = (dst_v0, dst_v1)
        rows_v = (rows_v0, rows_v1)
        sem_i = (sem_i0, sem_i1)
        sem_g = (sem_g0, sem_g1)
        core = lax.axis_index("c")
        sid = lax.axis_index("s")
        rows = pl.ds(sid * ROWS_PER_TILE, ROWS_PER_TILE)
        base = sid * EDGES_PER_TILE

        def zero_acc():
            pltpu.sync_copy(zfeat.at[rows], acc_sh.at[rows])

        def writeback(out_hbm):
            pltpu.sync_copy(acc_sh.at[rows], out_hbm.at[rows])

        def edge_slice(idx_hbm, c):
            return idx_hbm.at[pl.ds(base + c * CHUNK, CHUNK)]

        if with_counts:
            zero_acc()
            pltpu.sync_copy(ones128, ones_v)
            plsc.subcore_barrier()

            def cnt_loop(dst_hbm):
                def issue(c, b):
                    pltpu.async_copy(edge_slice(dst_hbm, c), dst_v[b], sem_i[b])

                def drain(c, b):
                    pltpu.make_async_copy(edge_slice(dst_hbm, c), dst_v[b],
                                          sem_i[b]).wait()
                    pltpu.sync_copy(ones_v, acc_sh.at[dst_v[b]], add=True)

                issue(0, 0)

                def outer(j, c):
                    for b in range(2):
                        i = 2 * j + b
                        issue(i + 1, 1 - b)
                        drain(i, b)
                    return c

                lax.fori_loop(0, (NCHUNK - 2) // 2, outer, 0)
                issue(NCHUNK - 1, 1)
                drain(NCHUNK - 2, 0)
                drain(NCHUNK - 1, 1)

            pl.when(core == 0)(lambda: cnt_loop(da))
            pl.when(core == 1)(lambda: cnt_loop(db))
            plsc.subcore_barrier()
            pl.when(core == 0)(lambda: writeback(cnt_a))
            pl.when(core == 1)(lambda: writeback(cnt_b))

        zero_acc()
        plsc.subcore_barrier()

        def run(src_hbm, dst_hbm, x_hbm):
            def issue_idx(c, b):
                pltpu.async_copy(edge_slice(src_hbm, c), src_v[b], sem_i[b])
                pltpu.async_copy(edge_slice(dst_hbm, c), dst_v[b], sem_i[b])

            def wait_idx(c, b):
                pltpu.make_async_copy(edge_slice(src_hbm, c), src_v[b],
                                      sem_i[b]).wait()
                pltpu.make_async_copy(edge_slice(dst_hbm, c), dst_v[b],
                                      sem_i[b]).wait()

            def issue_gather(b):
                pltpu.async_copy(x_hbm.at[src_v[b]], rows_v[b], sem_g[b])

            def drain_gather(x_hbm_, b):
                pltpu.make_async_copy(x_hbm_.at[src_v[b]], rows_v[b],
                                      sem_g[b]).wait()
                pltpu.sync_copy(rows_v[b], acc_sh.at[dst_v[b]], add=True)

            issue_idx(0, 0)
            issue_idx(1, 1)
            wait_idx(0, 0)
            issue_gather(0)

            def outer(j, c):
                for b in range(2):
                    i = 2 * j + b
                    nb = 1 - b
                    wait_idx(i + 1, nb)
                    issue_gather(nb)
                    drain_gather(x_hbm, b)
                    issue_idx(i + 2, b)
                return c

            lax.fori_loop(0, (NCHUNK - 2) // 2, outer, 0)
            i = NCHUNK - 2
            wait_idx(i + 1, 1)
            issue_gather(1)
            drain_gather(x_hbm, 0)
            drain_gather(x_hbm, 1)

        pl.when(core == 0)(lambda: run(sa, da, xa))
        pl.when(core == 1)(lambda: run(sb, db, xb))
        plsc.subcore_barrier()
        pl.when(core == 0)(lambda: writeback(agg_a))
        pl.when(core == 1)(lambda: writeback(agg_b))

    cnt_scratch = [
        pltpu.VMEM((CHUNK, D), jnp.float32),
    ] if with_counts else []
    return pl.kernel(
        body,
        out_type=out_type,
        mesh=_mesh,
        scratch_types=[pltpu.VMEM_SHARED((N_PAD, D), jnp.float32)]
        + cnt_scratch + [
            pltpu.VMEM((CHUNK,), jnp.int32),
            pltpu.VMEM((CHUNK,), jnp.int32),
            pltpu.VMEM((CHUNK,), jnp.int32),
            pltpu.VMEM((CHUNK,), jnp.int32),
            pltpu.VMEM((CHUNK, D), jnp.float32),
            pltpu.VMEM((CHUNK, D), jnp.float32),
            pltpu.SemaphoreType.DMA,
            pltpu.SemaphoreType.DMA,
            pltpu.SemaphoreType.DMA,
            pltpu.SemaphoreType.DMA,
        ],
    )


_sc_aggregate_l1 = _make_sc_aggregate(True)
_sc_aggregate_l2 = _make_sc_aggregate(False)


_BLK = 2000


def _mean(agg_ref, cnt_ref):
    return agg_ref[...] / jnp.maximum(cnt_ref[:, 0:1], 1.0)


def _sage(agg_ref, cnt_ref, x_ref, wl_ref, bl_ref, wr_ref):
    y = jnp.dot(_mean(agg_ref, cnt_ref), wl_ref[...],
                preferred_element_type=jnp.float32)
    y = y + bl_ref[...]
    return y + jnp.dot(x_ref[...], wr_ref[...],
                       preferred_element_type=jnp.float32)


_ROW_SPEC = pl.BlockSpec((_BLK, D), lambda i: (i, 0))
_W_SPEC = pl.BlockSpec((D, D), lambda i: (0, 0))
_B_SPEC = pl.BlockSpec((1, D), lambda i: (0, 0))
_SAGE_SPECS = [_ROW_SPEC, _ROW_SPEC, _ROW_SPEC, _W_SPEC, _B_SPEC, _W_SPEC]
_OUT_F32 = jax.ShapeDtypeStruct((N_NODES, D), jnp.float32)


def _layer1_call(args_t, args_c):
    def body(at, ct, xt, wlt, blt, wrt, ac, cc, xc, wlc, blc, wrc,
             ht_ref, hc_ref):
        ht_ref[...] = jnp.maximum(_sage(at, ct, xt, wlt, blt, wrt), 0.0)
        hc_ref[...] = jnp.maximum(_sage(ac, cc, xc, wlc, blc, wrc), 0.0)

    return pl.pallas_call(
        body,
        grid=(N_NODES // _BLK,),
        in_specs=_SAGE_SPECS + _SAGE_SPECS,
        out_specs=[_ROW_SPEC, _ROW_SPEC],
        out_shape=[_OUT_F32, _OUT_F32],
    )(*args_t, *args_c)


def _layer2_decode_call(args_t, args_c, Wd1T, bd1, Wd2T, bd2):
    def body(at, ct, xt, wlt, blt, wrt, ac, cc, xc, wlc, blc, wrc,
             wd1, bd1_, wd2, bd2_, zt_ref, zc_ref, r_ref):
        z = _sage(at, ct, xt, wlt, blt, wrt)
        zt_ref[...] = z
        zc_ref[...] = _sage(ac, cc, xc, wlc, blc, wrc)
        a = jnp.dot(z, wd1[...], preferred_element_type=jnp.float32)
        a = jnp.maximum(a + bd1_[...], 0.0)
        r_ref[...] = jnp.dot(a, wd2[...],
                             preferred_element_type=jnp.float32) + bd2_[...]

    return pl.pallas_call(
        body,
        grid=(N_NODES // _BLK,),
        in_specs=_SAGE_SPECS + _SAGE_SPECS + [
            pl.BlockSpec((D, 64), lambda i: (0, 0)),
            pl.BlockSpec((1, 64), lambda i: (0, 0)),
            pl.BlockSpec((64, D), lambda i: (0, 0)),
            _B_SPEC,
        ],
        out_specs=[_ROW_SPEC, _ROW_SPEC, _ROW_SPEC],
        out_shape=[_OUT_F32, _OUT_F32, _OUT_F32],
    )(*args_t, *args_c, Wd1T, bd1, Wd2T, bd2)


def kernel(x_transaction, x_client, edge_index_pays, edge_index_rev,
           Wl1_pt, bl1_pt, Wr1_pt, Wl1_tc, bl1_tc, Wr1_tc,
           Wl2_pt, bl2_pt, Wr2_pt, Wl2_tc, bl2_tc, Wr2_tc,
           Wd1, bd1, Wd2, bd2):
    f32 = jnp.float32
    xt = x_transaction.astype(f32)
    xc = x_client.astype(f32)
    eip = edge_index_pays.astype(jnp.int32)
    eir = edge_index_rev.astype(jnp.int32)
    sp, dp = eip[0], eip[1]
    sr, dr = eir[0], eir[1]
    zfeat = jnp.zeros((N_PAD, D), f32)
    ones128 = jnp.ones((CHUNK, D), f32)

    agg_t, agg_c, cnt_p, cnt_r = _sc_aggregate_l1(xc, xt, sp, dp, sr, dr,
                                                  zfeat, ones128)
    h_t, h_c = _layer1_call(
        (agg_t, cnt_p, xt, Wl1_pt.T, bl1_pt.reshape(1, -1), Wr1_pt.T),
        (agg_c, cnt_r, xc, Wl1_tc.T, bl1_tc.reshape(1, -1), Wr1_tc.T))

    agg2_t, agg2_c = _sc_aggregate_l2(h_c, h_t, sp, dp, sr, dr, zfeat,
                                      ones128)
    z_t, z_c, recon = _layer2_decode_call(
        (agg2_t, cnt_p, h_t, Wl2_pt.T, bl2_pt.reshape(1, -1), Wr2_pt.T),
        (agg2_c, cnt_r, h_c, Wl2_tc.T, bl2_tc.reshape(1, -1), Wr2_tc.T),
        Wd1.T, bd1.reshape(1, -1), Wd2.T, bd2.reshape(1, -1))
    return (recon, z_t, z_c)
